# Initial kernel scaffold; baseline (speedup 1.0000x reference)
import jax
import jax.numpy as jnp
from jax.experimental import pallas as pl


def kernel(features, feature_weights, intercept, slope, idx_i, idx_j):
    raise NotImplementedError("write your pallas kernel here")


# speedup vs baseline: 3.3822x; 3.3822x over previous
import functools
import jax
import jax.numpy as jnp
from jax import lax
from jax.experimental import pallas as pl
from jax.experimental.pallas import tpu as pltpu
from jax.experimental.pallas import tpu_sc as plsc

N_AUTH = 1_000_000
DIM = 16
N_PLANES = 4
BATCH = 16384
NC, NS, L = 2, 16, 16
NW = NC * NS
BPW = BATCH // NW
CHUNK = 128
NCHUNK = BPW // CHUNK


def _body(table, idx_i, idx_j, params, out,
          idxr, idxl, rows, pv, wsv, mat, outv,
          sem0, sem1, sem2, sem3):
    sems = (sem0, sem1, sem2, sem3)
    wid = lax.axis_index("s") * NC + lax.axis_index("c")
    base = wid * BPW

    pltpu.sync_copy(params, pv)
    ea = jnp.exp(pv[0, :])
    eb = jnp.exp(pv[1, :])
    wsv[...] = ea / (ea + eb)
    iota = lax.iota(jnp.int32, L)
    wvecs = [plsc.load_gather(wsv, [jnp.full((L,), c, jnp.int32)])
             for c in range(N_PLANES)]
    itcv = plsc.load_gather(pv, [jnp.full((L,), 2, jnp.int32),
                                 jnp.full((L,), 0, jnp.int32)])
    slpv = plsc.load_gather(pv, [jnp.full((L,), 2, jnp.int32),
                                 jnp.full((L,), 1, jnp.int32)])

    pltpu.sync_copy(idx_i.at[pl.ds(base, BPW)], idxr.at[0])
    pltpu.sync_copy(idx_j.at[pl.ds(base, BPW)], idxr.at[1])
    for s2 in range(2):
        for o in range(BPW // L):
            v = idxr[s2, pl.ds(o * L, L)]
            kc, kr = (o * L) // CHUNK, (o * L) % CHUNK
            for c in range(N_PLANES):
                vc = v if c == 0 else v + jnp.int32(c * N_AUTH)
                idxl[s2, c, kc, pl.ds(kr, L)] = vc

    handles = []
    for kc in range(NCHUNK):
        hs = []
        for s2 in range(2):
            for c in range(N_PLANES):
                hs.append(pltpu.async_copy(
                    table.at[idxl.at[s2, c, kc]],
                    rows.at[s2, c, kc], sems[kc]))
        handles.append(hs)

    for kc in range(NCHUNK):
        for h in handles[kc]:
            h.wait()

        def grp(g8, carry, kc=kc):
            kb = g8 * L
            for u in range(L):
                kr = kb + u
                acc = (rows[0, 0, kc, kr, :] - rows[1, 0, kc, kr, :]) * wvecs[0]
                for c in range(1, N_PLANES):
                    acc = acc + (rows[0, c, kc, kr, :]
                                 - rows[1, c, kc, kr, :]) * wvecs[c]
                mat[u, :] = acc * acc
            dv = plsc.load_gather(mat, [iota, jnp.full((L,), 0, jnp.int32)])
            for d in range(1, DIM):
                dv = dv + plsc.load_gather(
                    mat, [iota, jnp.full((L,), d, jnp.int32)])
            z = itcv - slpv * dv * (1.0 / DIM)
            outv[pl.ds(kc * CHUNK + kb, L)] = 1.0 / (1.0 + jnp.exp(-z))
            return carry

        lax.fori_loop(0, CHUNK // L, grp, 0)

    pltpu.sync_copy(outv, out.at[pl.ds(base, BPW)])


@functools.partial(jax.jit, static_argnames=())
def _run(table, idx_i, idx_j, params):
    mesh = plsc.VectorSubcoreMesh(core_axis_name="c", subcore_axis_name="s")
    f = functools.partial(
        pl.kernel,
        mesh=mesh,
        out_type=jax.ShapeDtypeStruct((BATCH,), jnp.float32),
        scratch_types=[
            pltpu.VMEM((2, BPW), jnp.int32),
            pltpu.VMEM((2, N_PLANES, NCHUNK, CHUNK), jnp.int32),
            pltpu.VMEM((2, N_PLANES, NCHUNK, CHUNK, DIM), jnp.float32),
            pltpu.VMEM((3, L), jnp.float32),
            pltpu.VMEM((L,), jnp.float32),
            pltpu.VMEM((L, L), jnp.float32),
            pltpu.VMEM((BPW,), jnp.float32),
            pltpu.SemaphoreType.DMA,
            pltpu.SemaphoreType.DMA,
            pltpu.SemaphoreType.DMA,
            pltpu.SemaphoreType.DMA,
        ],
        compiler_params=pltpu.CompilerParams(
            needs_layout_passes=False, use_tc_tiling_on_sc=False),
    )(_body)
    return f(table, idx_i, idx_j, params)


def kernel(features, feature_weights, intercept, slope, idx_i, idx_j):
    table = features.reshape(N_PLANES * N_AUTH, DIM)
    fw = feature_weights.reshape(-1).astype(jnp.float32)
    pad = jnp.zeros((L - 4,), jnp.float32)
    wa = jnp.concatenate([fw, pad])
    wb = jnp.concatenate([fw[1::2].reshape(2, 1),
                          fw[0::2].reshape(2, 1)], axis=1).reshape(-1)
    wb = jnp.concatenate([wb, pad])
    sc = jnp.concatenate([jnp.float32(intercept).reshape(1),
                          jnp.float32(slope).reshape(1),
                          jnp.zeros((L - 2,), jnp.float32)])
    params = jnp.stack([wa, wb, sc])
    return _run(table, idx_i.astype(jnp.int32), idx_j.astype(jnp.int32),
                params)


# speedup vs baseline: 9.4437x; 2.7922x over previous
import functools
import jax
import jax.numpy as jnp
from jax import lax
from jax.experimental import pallas as pl
from jax.experimental.pallas import tpu as pltpu
from jax.experimental.pallas import tpu_sc as plsc

N_AUTH = 1_000_000
DIM = 16
N_PLANES = 4
ROWW = N_PLANES * DIM
BATCH = 16384
NC, NS, L = 2, 16, 16
NW = NC * NS
BPW = BATCH // NW
CHUNK = 128
NCHUNK = BPW // CHUNK


def _body(table, idx_i, idx_j, params, out,
          idxl, rows, pv, wsv, mat, outv, sem0, sem1, sem2, sem3):
    sems = (sem0, sem1, sem2, sem3)
    wid = lax.axis_index("s") * NC + lax.axis_index("c")
    base = wid * BPW

    pltpu.sync_copy(params, pv)
    ea = jnp.exp(pv[0, :])
    eb = jnp.exp(pv[1, :])
    wsv[...] = ea / (ea + eb)
    iota = lax.iota(jnp.int32, L)
    wvecs = [plsc.load_gather(wsv, [jnp.full((L,), c, jnp.int32)])
             for c in range(N_PLANES)]
    itcv = plsc.load_gather(pv, [jnp.full((L,), 2, jnp.int32),
                                 jnp.full((L,), 0, jnp.int32)])
    slpv = plsc.load_gather(pv, [jnp.full((L,), 2, jnp.int32),
                                 jnp.full((L,), 1, jnp.int32)])

    for kc in range(NCHUNK):
        pltpu.sync_copy(idx_i.at[pl.ds(base + kc * CHUNK, CHUNK)],
                        idxl.at[0, kc])
        pltpu.sync_copy(idx_j.at[pl.ds(base + kc * CHUNK, CHUNK)],
                        idxl.at[1, kc])

    handles = []
    for kc in range(NCHUNK):
        hs = []
        for s2 in range(2):
            hs.append(pltpu.async_copy(
                table.at[idxl.at[s2, kc]], rows.at[s2, kc], sems[kc]))
        handles.append(hs)

    for kc in range(NCHUNK):
        for h in handles[kc]:
            h.wait()

        def grp(g8, carry, kc=kc):
            kb = g8 * L
            for u in range(L):
                kr = kb + u
                acc = None
                for c in range(N_PLANES):
                    t = (rows[0, kc, kr, pl.ds(c * DIM, DIM)]
                         - rows[1, kc, kr, pl.ds(c * DIM, DIM)]) * wvecs[c]
                    acc = t if acc is None else acc + t
                mat[u, :] = acc * acc
            dv = plsc.load_gather(mat, [iota, jnp.full((L,), 0, jnp.int32)])
            for d in range(1, DIM):
                dv = dv + plsc.load_gather(
                    mat, [iota, jnp.full((L,), d, jnp.int32)])
            z = itcv - slpv * dv * (1.0 / DIM)
            outv[pl.ds(kc * CHUNK + kb, L)] = 1.0 / (1.0 + jnp.exp(-z))
            return carry

        lax.fori_loop(0, CHUNK // L, grp, 0)

    pltpu.sync_copy(outv, out.at[pl.ds(base, BPW)])


@jax.jit
def _run(table, idx_i, idx_j, params):
    mesh = plsc.VectorSubcoreMesh(core_axis_name="c", subcore_axis_name="s")
    f = functools.partial(
        pl.kernel,
        mesh=mesh,
        out_type=jax.ShapeDtypeStruct((BATCH,), jnp.float32),
        scratch_types=[
            pltpu.VMEM((2, NCHUNK, CHUNK), jnp.int32),
            pltpu.VMEM((2, NCHUNK, CHUNK, ROWW), jnp.float32),
            pltpu.VMEM((3, L), jnp.float32),
            pltpu.VMEM((L,), jnp.float32),
            pltpu.VMEM((L, L), jnp.float32),
            pltpu.VMEM((BPW,), jnp.float32),
            pltpu.SemaphoreType.DMA,
            pltpu.SemaphoreType.DMA,
            pltpu.SemaphoreType.DMA,
            pltpu.SemaphoreType.DMA,
        ],
        compiler_params=pltpu.CompilerParams(
            needs_layout_passes=False, use_tc_tiling_on_sc=False),
    )(_body)
    return f(table, idx_i, idx_j, params)


def kernel(features, feature_weights, intercept, slope, idx_i, idx_j):
    table = jnp.transpose(features, (2, 0, 1, 3)).reshape(N_AUTH, ROWW)
    fw = feature_weights.reshape(-1).astype(jnp.float32)
    pad = jnp.zeros((L - 4,), jnp.float32)
    wa = jnp.concatenate([fw, pad])
    wb = jnp.concatenate([fw[1::2].reshape(2, 1),
                          fw[0::2].reshape(2, 1)], axis=1).reshape(-1)
    wb = jnp.concatenate([wb, pad])
    sc = jnp.concatenate([jnp.float32(intercept).reshape(1),
                          jnp.float32(slope).reshape(1),
                          jnp.zeros((L - 2,), jnp.float32)])
    params = jnp.stack([wa, wb, sc])
    return _run(table, idx_i.astype(jnp.int32), idx_j.astype(jnp.int32),
                params)


# speedup vs baseline: 11.3815x; 1.2052x over previous
import functools
import jax
import jax.numpy as jnp
from jax import lax
from jax.experimental import pallas as pl
from jax.experimental.pallas import tpu as pltpu
from jax.experimental.pallas import tpu_sc as plsc

N_AUTH = 1_000_000
DIM = 16
N_PLANES = 4
NROW = N_PLANES * DIM
BATCH = 16384
NC, NS, L = 2, 16, 16
NW = NC * NS
BPW = BATCH // NW
NG = BPW // L
BLK = 128
NBLK_FULL = N_AUTH // BLK
TAIL = N_AUTH - NBLK_FULL * BLK
APR = BLK // DIM
ETROWS = N_AUTH // APR


def _sweep_body(ft, params, etab, pv, wsv, vb0, vb1, ob, sa, sb):
    wid = lax.axis_index("s") * NC + lax.axis_index("c")
    pltpu.sync_copy(params, pv)
    ea = jnp.exp(pv[0, :])
    eb = jnp.exp(pv[1, :])
    wsv[...] = ea / (ea + eb)
    wvecs = [plsc.load_gather(wsv, [jnp.full((L,), c, jnp.int32)])
             for c in range(N_PLANES)]
    iota = lax.iota(jnp.int32, L)
    rowoff = iota // 8
    laneb = (iota % 8) * DIM

    nblk = jnp.where(wid < NBLK_FULL - (NBLK_FULL // NW) * NW,
                     NBLK_FULL // NW + 1, NBLK_FULL // NW)

    def blk_of(k):
        return wid + k * NW

    def fire(k, vb, sem):
        col = pl.multiple_of(blk_of(k) * BLK, BLK)
        pltpu.async_copy(ft.at[:, pl.ds(col, BLK)], vb, sem)

    def drain(vb, sem):
        pltpu.make_async_copy(ft.at[:, pl.ds(0, BLK)], vb, sem).wait()

    def combine(vb, nag):
        for d in range(DIM):
            lanes = laneb + d
            for ag in range(nag):
                acc = None
                for c in range(N_PLANES):
                    t = vb[c * DIM + d, pl.ds(ag * L, L)] * wvecs[c]
                    acc = t if acc is None else acc + t
                plsc.store_scatter(ob, [rowoff + 2 * ag, lanes], acc)

    def compute(k, vb):
        combine(vb, BLK // L)
        r0 = pl.multiple_of(blk_of(k) * DIM, DIM)
        pltpu.sync_copy(ob, etab.at[pl.ds(r0, DIM), :])

    fire(0, vb0, sa)

    def pair(kp, carry):
        k0 = kp * 2

        @pl.when(k0 + 1 < nblk)
        def _():
            fire(k0 + 1, vb1, sb)

        @pl.when(k0 < nblk)
        def _():
            drain(vb0, sa)
            compute(k0, vb0)

        @pl.when(k0 + 2 < nblk)
        def _():
            fire(k0 + 2, vb0, sa)

        @pl.when(k0 + 1 < nblk)
        def _():
            drain(vb1, sb)
            compute(k0 + 1, vb1)
        return carry

    lax.fori_loop(0, (NBLK_FULL // NW + 2) // 2, pair, 0)

    @pl.when(wid == NW - 1)
    def _():
        dcol = pl.multiple_of(wid * 0 + NBLK_FULL * BLK, BLK)
        pltpu.async_copy(ft.at[:, pl.ds(dcol, BLK)], vb0, sa)
        pltpu.make_async_copy(ft.at[:, pl.ds(0, BLK)], vb0, sa).wait()
        combine(vb0, TAIL // L)
        pltpu.sync_copy(ob.at[pl.ds(0, TAIL // APR)],
                        etab.at[pl.ds(NBLK_FULL * DIM, TAIL // APR), :])


def _gather_body(et, idx_i, idx_j, params, out,
                 idxr, idxl, rgb, pv, mat, outv, sa, sb):
    sems = (sa, sb)
    wid = lax.axis_index("s") * NC + lax.axis_index("c")
    base = wid * BPW
    iota = lax.iota(jnp.int32, L)

    pltpu.sync_copy(params, pv)
    itcv = plsc.load_gather(pv, [jnp.full((L,), 2, jnp.int32),
                                 jnp.full((L,), 0, jnp.int32)])
    slpv = plsc.load_gather(pv, [jnp.full((L,), 2, jnp.int32),
                                 jnp.full((L,), 1, jnp.int32)])

    pltpu.sync_copy(idx_i.at[pl.ds(base, BPW)], idxr.at[0])
    pltpu.sync_copy(idx_j.at[pl.ds(base, BPW)], idxr.at[1])

    def fire(g, b, sem):
        for s in range(2):
            nv = idxr[s, pl.ds(g * L, L)]
            idxl[b, s, :] = nv // APR
        for s in range(2):
            pltpu.async_copy(et.at[idxl.at[b, s]], rgb.at[b, s], sem)

    def drain(b, sem):
        for s in range(2):
            pltpu.make_async_copy(et.at[pl.ds(0, L), :],
                                  rgb.at[b, s], sem).wait()

    def compute(g, b):
        nvi = idxr[0, pl.ds(g * L, L)]
        nvj = idxr[1, pl.ds(g * L, L)]
        bb = jnp.full((L,), b, jnp.int32)
        s0 = jnp.full((L,), 0, jnp.int32)
        s1 = jnp.full((L,), 1, jnp.int32)
        for u in range(L):
            uu = jnp.full((L,), u, jnp.int32)
            zi = plsc.load_gather(
                rgb, [bb, s0, uu, iota + (nvi[u] % APR) * DIM])
            zj = plsc.load_gather(
                rgb, [bb, s1, uu, iota + (nvj[u] % APR) * DIM])
            dd = zi - zj
            mat[u, :] = dd * dd
        dv = plsc.load_gather(mat, [iota, jnp.full((L,), 0, jnp.int32)])
        for d in range(1, DIM):
            dv = dv + plsc.load_gather(
                mat, [iota, jnp.full((L,), d, jnp.int32)])
        z = itcv - slpv * dv * (1.0 / DIM)
        outv[pl.ds(g * L, L)] = 1.0 / (1.0 + jnp.exp(-z))

    fire(0, 0, sems[0])

    def pair(gp, carry):
        g0 = gp * 2
        fire(g0 + 1, 1, sems[1])
        drain(0, sems[0])
        compute(g0, 0)
        fire(g0 + 2, 0, sems[0])
        drain(1, sems[1])
        compute(g0 + 1, 1)
        return carry

    lax.fori_loop(0, NG // 2 - 1, pair, 0)
    fire(NG - 1, 1, sems[1])
    drain(0, sems[0])
    compute(NG - 2, 0)
    drain(1, sems[1])
    compute(NG - 1, 1)

    pltpu.sync_copy(outv, out.at[pl.ds(base, BPW)])


@jax.jit
def _run(ft, idx_i, idx_j, params):
    mesh = plsc.VectorSubcoreMesh(core_axis_name="c", subcore_axis_name="s")
    sweep = functools.partial(
        pl.kernel,
        mesh=mesh,
        out_type=jax.ShapeDtypeStruct((ETROWS, BLK), jnp.float32),
        scratch_types=[
            pltpu.VMEM((3, L), jnp.float32),
            pltpu.VMEM((L,), jnp.float32),
            pltpu.VMEM((NROW, BLK), jnp.float32),
            pltpu.VMEM((NROW, BLK), jnp.float32),
            pltpu.VMEM((DIM, BLK), jnp.float32),
            pltpu.SemaphoreType.DMA,
            pltpu.SemaphoreType.DMA,
        ],
        compiler_params=pltpu.CompilerParams(
            needs_layout_passes=False, use_tc_tiling_on_sc=True),
    )(_sweep_body)
    etab = sweep(ft, params)

    gather = functools.partial(
        pl.kernel,
        mesh=mesh,
        out_type=jax.ShapeDtypeStruct((BATCH,), jnp.float32),
        scratch_types=[
            pltpu.VMEM((2, BPW), jnp.int32),
            pltpu.VMEM((2, 2, L), jnp.int32),
            pltpu.VMEM((2, 2, L, BLK), jnp.float32),
            pltpu.VMEM((3, L), jnp.float32),
            pltpu.VMEM((L, L), jnp.float32),
            pltpu.VMEM((BPW,), jnp.float32),
            pltpu.SemaphoreType.DMA,
            pltpu.SemaphoreType.DMA,
        ],
        compiler_params=pltpu.CompilerParams(
            needs_layout_passes=False, use_tc_tiling_on_sc=True),
    )(_gather_body)
    return gather(etab, idx_i, idx_j, params)


def kernel(features, feature_weights, intercept, slope, idx_i, idx_j):
    ft = jnp.transpose(features, (0, 1, 3, 2)).reshape(NROW, N_AUTH)
    fw = feature_weights.reshape(-1).astype(jnp.float32)
    pad = jnp.zeros((L - 4,), jnp.float32)
    wa = jnp.concatenate([fw, pad])
    wb = jnp.concatenate([fw[1::2].reshape(2, 1),
                          fw[0::2].reshape(2, 1)], axis=1).reshape(-1)
    wb = jnp.concatenate([wb, pad])
    sc = jnp.concatenate([jnp.float32(intercept).reshape(1),
                          jnp.float32(slope).reshape(1),
                          jnp.zeros((L - 2,), jnp.float32)])
    params = jnp.stack([wa, wb, sc])
    return _run(ft, idx_i.astype(jnp.int32), idx_j.astype(jnp.int32),
                params)


# speedup vs baseline: 16.5899x; 1.4576x over previous
import functools
import jax
import jax.numpy as jnp
from jax import lax
from jax.experimental import pallas as pl
from jax.experimental.pallas import tpu as pltpu
from jax.experimental.pallas import tpu_sc as plsc

N_AUTH = 1_000_000
DIM = 16
N_PLANES = 4
NROW = N_PLANES * DIM
BATCH = 16384
NC, NS, L = 2, 16, 16
NW = NC * NS
BPW = BATCH // NW
NG = BPW // L
BLK = 128
NBLK_FULL = N_AUTH // BLK
BPT = NBLK_FULL // NW
NEXTRA = NBLK_FULL - BPT * NW
TAIL = N_AUTH - NBLK_FULL * BLK
ETROWS = (NBLK_FULL + 1) * DIM
GROWS = ETROWS * BLK // L


def _sweep_body(ft, params, etab,
                pv, wsv, vb0, vb1, ob0, ob1, obt,
                sa, sb, oa, obs, st):
    wid = lax.axis_index("s") * NC + lax.axis_index("c")
    pltpu.sync_copy(params, pv)
    ea = jnp.exp(pv[0, :])
    eb = jnp.exp(pv[1, :])
    wsv[...] = ea / (ea + eb)
    wvecs = [plsc.load_gather(wsv, [jnp.full((L,), c, jnp.int32)])
             for c in range(N_PLANES)]

    def blk_of(k):
        return wid + k * NW

    def fire_in(k, vb, sem):
        col = pl.multiple_of(blk_of(k) * BLK, BLK)
        pltpu.async_copy(ft.at[:, pl.ds(col, BLK)], vb, sem)

    def drain_in(vb, sem):
        pltpu.make_async_copy(ft.at[:, pl.ds(0, BLK)], vb, sem).wait()

    def combine(vb, ob, nag):
        for d in range(DIM):
            for ag in range(nag):
                sl = pl.ds(ag * L, L)
                a01 = vb[d, sl] * wvecs[0] + vb[DIM + d, sl] * wvecs[1]
                a23 = (vb[2 * DIM + d, sl] * wvecs[2]
                       + vb[3 * DIM + d, sl] * wvecs[3])
                ob[d, sl] = a01 + a23

    def fire_out(k, ob, sem):
        r0 = pl.multiple_of(blk_of(k) * DIM, DIM)
        pltpu.async_copy(ob, etab.at[pl.ds(r0, DIM), :], sem)

    def drain_out(ob, sem):
        pltpu.make_async_copy(ob, etab.at[pl.ds(0, DIM), :], sem).wait()

    bufs = ((vb0, ob0, sa, oa), (vb1, ob1, sb, obs))

    def step(k, parity, first):
        vb, ob, sin, sout = bufs[parity]
        if not first:
            drain_out(ob, sout)
        drain_in(vb, sin)
        combine(vb, ob, BLK // L)
        fire_out(k, ob, sout)

    fire_in(0, vb0, sa)
    fire_in(1, vb1, sb)
    step(0, 0, True)
    fire_in(2, vb0, sa)
    step(1, 1, True)
    fire_in(3, vb1, sb)

    def pair(kp, carry):
        k0 = kp * 2
        step(k0, 0, False)

        @pl.when(k0 + 2 < BPT)
        def _():
            fire_in(k0 + 2, vb0, sa)
        step(k0 + 1, 1, False)

        @pl.when(k0 + 3 < BPT)
        def _():
            fire_in(k0 + 3, vb1, sb)
        return carry

    lax.fori_loop(1, BPT // 2, pair, 0)

    @pl.when(wid < NEXTRA)
    def _():
        fire_in(BPT, vb0, sa)
        drain_out(ob0, oa)
        drain_in(vb0, sa)
        combine(vb0, ob0, BLK // L)
        fire_out(BPT, ob0, oa)

    @pl.when(wid == NW - 1)
    def _():
        dcol = pl.multiple_of(wid * 0 + NBLK_FULL * BLK, BLK)
        pltpu.async_copy(ft.at[:, pl.ds(dcol, BLK)], vb0, sa)
        pltpu.make_async_copy(ft.at[:, pl.ds(0, BLK)], vb0, sa).wait()
        combine(vb0, obt, BLK // L)
        pltpu.async_copy(obt, etab.at[pl.ds(NBLK_FULL * DIM, DIM), :], st)
        pltpu.make_async_copy(obt, etab.at[pl.ds(0, DIM), :], st).wait()

    drain_out(ob0, oa)
    drain_out(ob1, obs)


def _gather_body(gt, idx_i, idx_j, params, out,
                 idxr, idxl, rgb, pv, mat, outv, sa, sb):
    sems = (sa, sb)
    wid = lax.axis_index("s") * NC + lax.axis_index("c")
    base = wid * BPW
    iota = lax.iota(jnp.int32, L)

    pltpu.sync_copy(params, pv)
    itcv = plsc.load_gather(pv, [jnp.full((L,), 2, jnp.int32),
                                 jnp.full((L,), 0, jnp.int32)])
    slpv = plsc.load_gather(pv, [jnp.full((L,), 2, jnp.int32),
                                 jnp.full((L,), 1, jnp.int32)])

    pltpu.sync_copy(idx_i.at[pl.ds(base, BPW)], idxr.at[0])
    pltpu.sync_copy(idx_j.at[pl.ds(base, BPW)], idxr.at[1])

    def fire(g, b, sem):
        for s in range(2):
            nv = idxr[s, pl.ds(g * L, L)]
            bv = (nv // BLK) * BLK + (nv % BLK) // L
            for u in range(L):
                gr = iota * 8 + jnp.full((L,), bv[u], jnp.int32)
                idxl[b, s, u // 8, pl.ds((u % 8) * L, L)] = gr
        for s in range(2):
            for c in range(2):
                pltpu.async_copy(gt.at[idxl.at[b, s, c]],
                                 rgb.at[b, s, c], sem)

    def drain(b, sem):
        for s in range(2):
            for c in range(2):
                pltpu.make_async_copy(gt.at[pl.ds(0, 8 * L)],
                                      rgb.at[b, s, c], sem).wait()

    def compute(g, b):
        nvi = idxr[0, pl.ds(g * L, L)]
        nvj = idxr[1, pl.ds(g * L, L)]
        li = nvi % L
        lj = nvj % L
        bb = jnp.full((L,), b, jnp.int32)
        s0 = jnp.full((L,), 0, jnp.int32)
        s1 = jnp.full((L,), 1, jnp.int32)
        for u in range(L):
            cc = jnp.full((L,), u // 8, jnp.int32)
            rows = iota + (u % 8) * L
            zi = plsc.load_gather(
                rgb, [bb, s0, cc, rows, jnp.full((L,), li[u], jnp.int32)])
            zj = plsc.load_gather(
                rgb, [bb, s1, cc, rows, jnp.full((L,), lj[u], jnp.int32)])
            dd = zi - zj
            mat[u, :] = dd * dd
        dv = plsc.load_gather(mat, [iota, jnp.full((L,), 0, jnp.int32)])
        for d in range(1, DIM):
            dv = dv + plsc.load_gather(
                mat, [iota, jnp.full((L,), d, jnp.int32)])
        z = itcv - slpv * dv * (1.0 / DIM)
        outv[pl.ds(g * L, L)] = 1.0 / (1.0 + jnp.exp(-z))

    fire(0, 0, sems[0])

    def pair(gp, carry):
        g0 = gp * 2
        fire(g0 + 1, 1, sems[1])
        drain(0, sems[0])
        compute(g0, 0)
        fire(g0 + 2, 0, sems[0])
        drain(1, sems[1])
        compute(g0 + 1, 1)
        return carry

    lax.fori_loop(0, NG // 2 - 1, pair, 0)
    fire(NG - 1, 1, sems[1])
    drain(0, sems[0])
    compute(NG - 2, 0)
    drain(1, sems[1])
    compute(NG - 1, 1)

    pltpu.sync_copy(outv, out.at[pl.ds(base, BPW)])


@jax.jit
def _run(ft, idx_i, idx_j, params):
    mesh = plsc.VectorSubcoreMesh(core_axis_name="c", subcore_axis_name="s")
    sweep = functools.partial(
        pl.kernel,
        mesh=mesh,
        out_type=jax.ShapeDtypeStruct((ETROWS, BLK), jnp.float32),
        scratch_types=[
            pltpu.VMEM((3, L), jnp.float32),
            pltpu.VMEM((L,), jnp.float32),
            pltpu.VMEM((NROW, BLK), jnp.float32),
            pltpu.VMEM((NROW, BLK), jnp.float32),
            pltpu.VMEM((DIM, BLK), jnp.float32),
            pltpu.VMEM((DIM, BLK), jnp.float32),
            pltpu.VMEM((DIM, BLK), jnp.float32),
            pltpu.SemaphoreType.DMA,
            pltpu.SemaphoreType.DMA,
            pltpu.SemaphoreType.DMA,
            pltpu.SemaphoreType.DMA,
            pltpu.SemaphoreType.DMA,
        ],
        compiler_params=pltpu.CompilerParams(
            needs_layout_passes=False, use_tc_tiling_on_sc=True),
    )(_sweep_body)
    etab = sweep(ft, params)

    gt = etab.reshape(GROWS, DIM)
    gather = functools.partial(
        pl.kernel,
        mesh=mesh,
        out_type=jax.ShapeDtypeStruct((BATCH,), jnp.float32),
        scratch_types=[
            pltpu.VMEM((2, BPW), jnp.int32),
            pltpu.VMEM((2, 2, 2, 8 * L), jnp.int32),
            pltpu.VMEM((2, 2, 2, 8 * L, L), jnp.float32),
            pltpu.VMEM((3, L), jnp.float32),
            pltpu.VMEM((L, L), jnp.float32),
            pltpu.VMEM((BPW,), jnp.float32),
            pltpu.SemaphoreType.DMA,
            pltpu.SemaphoreType.DMA,
        ],
        compiler_params=pltpu.CompilerParams(
            needs_layout_passes=False, use_tc_tiling_on_sc=False),
    )(_gather_body)
    return gather(gt, idx_i, idx_j, params)


def kernel(features, feature_weights, intercept, slope, idx_i, idx_j):
    ft = jnp.transpose(features, (0, 1, 3, 2)).reshape(NROW, N_AUTH)
    fw = feature_weights.reshape(-1).astype(jnp.float32)
    pad = jnp.zeros((L - 4,), jnp.float32)
    wa = jnp.concatenate([fw, pad])
    wb = jnp.concatenate([fw[1::2].reshape(2, 1),
                          fw[0::2].reshape(2, 1)], axis=1).reshape(-1)
    wb = jnp.concatenate([wb, pad])
    sc = jnp.concatenate([jnp.float32(intercept).reshape(1),
                          jnp.float32(slope).reshape(1),
                          jnp.zeros((L - 2,), jnp.float32)])
    params = jnp.stack([wa, wb, sc])
    return _run(ft, idx_i.astype(jnp.int32), idx_j.astype(jnp.int32),
                params)
